# Initial kernel scaffold; baseline (speedup 1.0000x reference)
import functools

import jax
import jax.numpy as jnp
from jax import lax
from jax.experimental import pallas as pl
from jax.experimental.pallas import tpu as pltpu
from jax.experimental.pallas import tpu_sc as plsc

_LANES = 128
_K = 8
_CHUNK = _K * _LANES


@functools.lru_cache(maxsize=None)
def _build(N, D):
    info = plsc.get_sparse_core_info()
    nw = info.num_cores * info.num_subcores
    n_per_w = N // nw
    n_chunks = n_per_w // _CHUNK
    assert n_per_w % _CHUNK == 0, (N, nw, _CHUNK)

    mesh = plsc.VectorSubcoreMesh(core_axis_name="c", subcore_axis_name="s")

    @functools.partial(
        pl.kernel,
        mesh=mesh,
        out_type=jax.ShapeDtypeStruct((N, D), jnp.float32),
        scratch_types=[
            pltpu.VMEM((_K, _LANES), jnp.int32),
            pltpu.VMEM((_CHUNK, D), jnp.float32),
            pltpu.SemaphoreType.DMA,
        ],
    )
    def gather_kernel(idx_hbm, table_hbm, out_hbm, idx_v, rows_v, sem):
        wid = lax.axis_index("s") * info.num_cores + lax.axis_index("c")
        row_base = wid * (n_per_w // _LANES)

        def body(g, carry):
            row_off = row_base + g * _K
            pltpu.sync_copy(idx_hbm.at[pl.ds(row_off, _K)], idx_v)
            copies = [
                pltpu.async_copy(
                    table_hbm.at[idx_v.at[j]],
                    rows_v.at[pl.ds(j * _LANES, _LANES)],
                    sem,
                )
                for j in range(_K)
            ]
            for cp in copies:
                cp.wait()
            out_off = wid * n_per_w + g * _CHUNK
            pltpu.sync_copy(rows_v, out_hbm.at[pl.ds(out_off, _CHUNK)])
            return carry

        lax.fori_loop(0, n_chunks, body, 0)

    return gather_kernel


def kernel(X, table):
    B, H = X.shape
    V, D = table.shape
    N = B * H
    idx = X.reshape(N // _LANES, _LANES).astype(jnp.int32)
    out = _build(N, D)(idx, table)
    return out.reshape(B, H, D)


# speedup vs baseline: 1.0942x; 1.0942x over previous
import functools

import jax
import jax.numpy as jnp
from jax import lax
from jax.experimental import pallas as pl
from jax.experimental.pallas import tpu as pltpu
from jax.experimental.pallas import tpu_sc as plsc

_LANES = 128
_K = 8
_CHUNK = _K * _LANES


@functools.lru_cache(maxsize=None)
def _build(N, D):
    info = plsc.get_sparse_core_info()
    nw = info.num_cores * info.num_subcores
    n_per_w = N // nw
    n_chunks = n_per_w // _CHUNK
    assert n_per_w % _CHUNK == 0, (N, nw, _CHUNK)

    mesh = plsc.VectorSubcoreMesh(core_axis_name="c", subcore_axis_name="s")

    @functools.partial(
        pl.kernel,
        mesh=mesh,
        compiler_params=pltpu.CompilerParams(use_tc_tiling_on_sc=False),
        out_type=jax.ShapeDtypeStruct((N, D), jnp.float32),
        scratch_types=[
            pltpu.VMEM((_K, _LANES), jnp.int32),
            pltpu.VMEM((_CHUNK, D), jnp.float32),
            pltpu.SemaphoreType.DMA,
        ],
    )
    def gather_kernel(idx_hbm, table_hbm, out_hbm, idx_v, rows_v, sem):
        wid = lax.axis_index("s") * info.num_cores + lax.axis_index("c")
        row_base = wid * (n_per_w // _LANES)

        def body(g, carry):
            row_off = row_base + g * _K
            pltpu.sync_copy(idx_hbm.at[pl.ds(row_off, _K)], idx_v)
            copies = [
                pltpu.async_copy(
                    table_hbm.at[idx_v.at[j]],
                    rows_v.at[pl.ds(j * _LANES, _LANES)],
                    sem,
                )
                for j in range(_K)
            ]
            for cp in copies:
                cp.wait()
            out_off = wid * n_per_w + g * _CHUNK
            pltpu.sync_copy(rows_v, out_hbm.at[pl.ds(out_off, _CHUNK)])
            return carry

        lax.fori_loop(0, n_chunks, body, 0)

    return gather_kernel


def kernel(X, table):
    B, H = X.shape
    V, D = table.shape
    N = B * H
    idx = X.reshape(N // _LANES, _LANES).astype(jnp.int32)
    out = _build(N, D)(idx, table)
    return out.reshape(B, H, D)


# speedup vs baseline: 1.1134x; 1.0176x over previous
import functools

import jax
import jax.numpy as jnp
from jax import lax
from jax.experimental import pallas as pl
from jax.experimental.pallas import tpu as pltpu
from jax.experimental.pallas import tpu_sc as plsc

_LANES = 128
_K = 8
_CHUNK = _K * _LANES


@functools.lru_cache(maxsize=None)
def _build(N, D):
    info = plsc.get_sparse_core_info()
    nw = info.num_cores * info.num_subcores
    n_per_w = N // nw
    n_chunks = n_per_w // _CHUNK
    assert n_per_w % _CHUNK == 0, (N, nw, _CHUNK)

    mesh = plsc.VectorSubcoreMesh(core_axis_name="c", subcore_axis_name="s")

    @functools.partial(
        pl.kernel,
        mesh=mesh,
        compiler_params=pltpu.CompilerParams(use_tc_tiling_on_sc=False),
        out_type=jax.ShapeDtypeStruct((N, D), jnp.float32),
        scratch_types=[
            pltpu.VMEM((2, _K, _LANES), jnp.int32),
            pltpu.VMEM((2, _CHUNK, D), jnp.float32),
            pltpu.SemaphoreType.DMA((2,)),
            pltpu.SemaphoreType.DMA((2,)),
        ],
    )
    def gather_kernel(idx_hbm, table_hbm, out_hbm, idx_v, rows_v, gsem, wsem):
        wid = lax.axis_index("s") * info.num_cores + lax.axis_index("c")
        row_base = wid * (n_per_w // _LANES)
        out_base = wid * n_per_w

        def load_idx(g, b):
            pltpu.sync_copy(idx_hbm.at[pl.ds(row_base + g * _K, _K)],
                            idx_v.at[b])

        def fire_gathers(g, b):
            for j in range(_K):
                pltpu.async_copy(
                    table_hbm.at[idx_v.at[b, j]],
                    rows_v.at[b, pl.ds(j * _LANES, _LANES)],
                    gsem.at[b],
                )

        def wait_gathers(b):
            pltpu.make_async_copy(out_hbm.at[pl.ds(0, _CHUNK)],
                                  rows_v.at[b], gsem.at[b]).wait()

        def fire_writeback(g, b):
            pltpu.async_copy(rows_v.at[b],
                             out_hbm.at[pl.ds(out_base + g * _CHUNK, _CHUNK)],
                             wsem.at[b])

        def wait_writeback(b):
            pltpu.make_async_copy(rows_v.at[b],
                                  out_hbm.at[pl.ds(0, _CHUNK)],
                                  wsem.at[b]).wait()

        load_idx(0, 0)
        fire_gathers(0, 0)

        def body(g, carry):
            b = lax.rem(g, 2)
            nb = 1 - b

            @pl.when(g < n_chunks - 1)
            def _():
                load_idx(g + 1, nb)

                @pl.when(g >= 1)
                def _():
                    wait_writeback(nb)

                fire_gathers(g + 1, nb)

            wait_gathers(b)
            fire_writeback(g, b)
            return carry

        lax.fori_loop(0, n_chunks, body, 0)

        last = n_chunks - 1
        wait_writeback(last % 2)
        if n_chunks >= 2:
            wait_writeback(1 - last % 2)

    return gather_kernel


def kernel(X, table):
    B, H = X.shape
    V, D = table.shape
    N = B * H
    idx = X.reshape(N // _LANES, _LANES).astype(jnp.int32)
    out = _build(N, D)(idx, table)
    return out.reshape(B, H, D)
